# Initial kernel scaffold; baseline (speedup 1.0000x reference)
import jax
import jax.numpy as jnp
from jax.experimental import pallas as pl


def kernel(x, token_table, pos_table):
    raise NotImplementedError("write your pallas kernel here")


# speedup vs baseline: 2.3333x; 2.3333x over previous
import functools

import jax
import jax.numpy as jnp
from jax import lax
from jax.experimental import pallas as pl
from jax.experimental.pallas import tpu as pltpu
from jax.experimental.pallas import tpu_sc as plsc

BATCH = 1024
MAXLEN = 200
EMBED = 64
LANES = 16

NUM_CORES = 2
NUM_SUBCORES = 16
NW = NUM_CORES * NUM_SUBCORES

N_TOTAL = BATCH * MAXLEN
N_PER_W = N_TOTAL // NW
CHUNK = MAXLEN
N_CHUNKS = N_PER_W // CHUNK
VECS_PER_CHUNK = CHUNK * EMBED // LANES


def _body(x_hbm, tok_hbm, pos_hbm, out_hbm,
          idx_v, pos_v, rows0, rows1, sem0, sem1):
  wid = lax.axis_index("s") * NUM_CORES + lax.axis_index("c")
  base = wid * N_PER_W

  pltpu.sync_copy(x_hbm.at[pl.ds(base, N_PER_W)], idx_v)
  pltpu.sync_copy(pos_hbm, pos_v)

  rows = (rows0, rows1)
  sems = (sem0, sem1)

  def start(c, b):
    pltpu.async_copy(
        tok_hbm.at[idx_v.at[pl.ds(c * CHUNK, CHUNK)]], rows[b], sems[b])

  def wait(b):
    pltpu.make_async_copy(tok_hbm.at[pl.ds(0, CHUNK)], rows[b], sems[b]).wait()

  start(0, 0)
  start(1, 1)

  def add_pos(rref):
    def body(i, _):
      r = i // (EMBED // LANES)
      j = (i % (EMBED // LANES)) * LANES
      rref[r, pl.ds(j, LANES)] += pos_v[r, pl.ds(j, LANES)]
      return 0
    lax.fori_loop(0, VECS_PER_CHUNK, body, 0)

  def outer(c0, _):
    for b in range(2):
      c = c0 + b
      wait(b)
      add_pos(rows[b])
      pltpu.sync_copy(rows[b], out_hbm.at[pl.ds(base + c * CHUNK, CHUNK)])

      @pl.when(c + 2 < N_CHUNKS)
      def _():
        start(c + 2, b)
    return 0

  lax.fori_loop(0, N_CHUNKS // 2, lambda i, s: outer(i * 2, s), 0, unroll=False)


@jax.jit
def _tok_pos_embed(x_flat, token_table, pos_table):
  mesh = plsc.VectorSubcoreMesh(core_axis_name="c", subcore_axis_name="s")
  kern = functools.partial(
      pl.kernel,
      out_type=jax.ShapeDtypeStruct((N_TOTAL, EMBED), jnp.float32),
      mesh=mesh,
      scratch_types=[
          pltpu.VMEM((N_PER_W,), jnp.int32),
          pltpu.VMEM((MAXLEN, EMBED), jnp.float32),
          pltpu.VMEM((CHUNK, EMBED), jnp.float32),
          pltpu.VMEM((CHUNK, EMBED), jnp.float32),
          pltpu.SemaphoreType.DMA,
          pltpu.SemaphoreType.DMA,
      ],
      compiler_params=pltpu.CompilerParams(use_tc_tiling_on_sc=False),
  )(_body)
  return kern(x_flat, token_table, pos_table)


def kernel(x, token_table, pos_table):
  x_flat = x.reshape(-1).astype(jnp.int32)
  out = _tok_pos_embed(x_flat, token_table, pos_table)
  return out.reshape(BATCH, MAXLEN, EMBED)


# speedup vs baseline: 3.1087x; 1.3323x over previous
import functools

import jax
import jax.numpy as jnp
from jax import lax
from jax.experimental import pallas as pl
from jax.experimental.pallas import tpu as pltpu
from jax.experimental.pallas import tpu_sc as plsc

BATCH = 1024
MAXLEN = 200
EMBED = 64
LANES = 16

NUM_CORES = 2
NUM_SUBCORES = 16
NW = NUM_CORES * NUM_SUBCORES

N_TOTAL = BATCH * MAXLEN
N_PER_W = N_TOTAL // NW
CHUNK = MAXLEN
N_CHUNKS = N_PER_W // CHUNK
VECS_PER_CHUNK = CHUNK * EMBED // LANES


def _body(x_hbm, tok_hbm, pos_hbm, out_hbm,
          idx_v, pos_v, rows0, rows1, sem0, sem1):
  wid = lax.axis_index("s") * NUM_CORES + lax.axis_index("c")
  base = wid * N_PER_W

  pltpu.sync_copy(x_hbm.at[pl.ds(base, N_PER_W)], idx_v)
  pltpu.sync_copy(pos_hbm, pos_v)

  rows = (rows0, rows1)
  sems = (sem0, sem1)

  def start(c, b):
    pltpu.async_copy(
        tok_hbm.at[idx_v.at[pl.ds(c * CHUNK, CHUNK)]], rows[b], sems[b])

  def wait(b):
    pltpu.make_async_copy(tok_hbm.at[pl.ds(0, CHUNK)], rows[b], sems[b]).wait()

  start(0, 0)
  start(1, 1)

  def add_pos(rref):
    @plsc.parallel_loop(0, CHUNK, 1, unroll=4)
    def _(r):
      for j in range(EMBED // LANES):
        rref[r, pl.ds(j * LANES, LANES)] += pos_v[r, pl.ds(j * LANES, LANES)]

  def outer(c0, _):
    for b in range(2):
      c = c0 + b
      wait(b)
      add_pos(rows[b])
      pltpu.sync_copy(rows[b], out_hbm.at[pl.ds(base + c * CHUNK, CHUNK)])

      @pl.when(c + 2 < N_CHUNKS)
      def _():
        start(c + 2, b)
    return 0

  lax.fori_loop(0, N_CHUNKS // 2, lambda i, s: outer(i * 2, s), 0, unroll=False)


@jax.jit
def _tok_pos_embed(x_flat, token_table, pos_table):
  mesh = plsc.VectorSubcoreMesh(core_axis_name="c", subcore_axis_name="s")
  kern = functools.partial(
      pl.kernel,
      out_type=jax.ShapeDtypeStruct((N_TOTAL, EMBED), jnp.float32),
      mesh=mesh,
      scratch_types=[
          pltpu.VMEM((N_PER_W,), jnp.int32),
          pltpu.VMEM((MAXLEN, EMBED), jnp.float32),
          pltpu.VMEM((CHUNK, EMBED), jnp.float32),
          pltpu.VMEM((CHUNK, EMBED), jnp.float32),
          pltpu.SemaphoreType.DMA,
          pltpu.SemaphoreType.DMA,
      ],
      compiler_params=pltpu.CompilerParams(use_tc_tiling_on_sc=False),
  )(_body)
  return kern(x_flat, token_table, pos_table)


def kernel(x, token_table, pos_table):
  x_flat = x.reshape(-1).astype(jnp.int32)
  out = _tok_pos_embed(x_flat, token_table, pos_table)
  return out.reshape(BATCH, MAXLEN, EMBED)


# speedup vs baseline: 3.1747x; 1.0212x over previous
import functools

import jax
import jax.numpy as jnp
from jax import lax
from jax.experimental import pallas as pl
from jax.experimental.pallas import tpu as pltpu
from jax.experimental.pallas import tpu_sc as plsc

BATCH = 1024
MAXLEN = 200
EMBED = 64
LANES = 16

NUM_CORES = 2
NUM_SUBCORES = 16
NW = NUM_CORES * NUM_SUBCORES

N_TOTAL = BATCH * MAXLEN
N_PER_W = N_TOTAL // NW
CHUNK = MAXLEN
N_CHUNKS = N_PER_W // CHUNK
VECS_PER_CHUNK = CHUNK * EMBED // LANES


def _body(x_hbm, tok_hbm, pos_hbm, out_hbm,
          idx_v, pos_sh, rows0, rows1, sem0, sem1):
  sid = lax.axis_index("s")
  wid = sid * NUM_CORES + lax.axis_index("c")
  base = wid * N_PER_W

  pltpu.sync_copy(x_hbm.at[pl.ds(base, N_PER_W)], idx_v)

  @pl.when(sid == 0)
  def _():
    pltpu.sync_copy(pos_hbm, pos_sh)

  plsc.subcore_barrier()

  rows = (rows0, rows1)
  sems = (sem0, sem1)

  def start(c, b):
    pltpu.sync_copy(pos_sh, rows[b])
    pltpu.async_copy(
        tok_hbm.at[idx_v.at[pl.ds(c * CHUNK, CHUNK)]], rows[b], sems[b],
        add=True)

  def wait(b):
    pltpu.make_async_copy(tok_hbm.at[pl.ds(0, CHUNK)], rows[b], sems[b]).wait()

  start(0, 0)
  start(1, 1)

  def outer(c0, _):
    for b in range(2):
      c = c0 + b
      wait(b)
      pltpu.sync_copy(rows[b], out_hbm.at[pl.ds(base + c * CHUNK, CHUNK)])

      @pl.when(c + 2 < N_CHUNKS)
      def _():
        start(c + 2, b)
    return 0

  lax.fori_loop(0, N_CHUNKS // 2, lambda i, s: outer(i * 2, s), 0, unroll=False)


@jax.jit
def _tok_pos_embed(x_flat, token_table, pos_table):
  mesh = plsc.VectorSubcoreMesh(core_axis_name="c", subcore_axis_name="s")
  kern = functools.partial(
      pl.kernel,
      out_type=jax.ShapeDtypeStruct((N_TOTAL, EMBED), jnp.float32),
      mesh=mesh,
      scratch_types=[
          pltpu.VMEM((N_PER_W,), jnp.int32),
          pltpu.VMEM_SHARED((MAXLEN, EMBED), jnp.float32),
          pltpu.VMEM((CHUNK, EMBED), jnp.float32),
          pltpu.VMEM((CHUNK, EMBED), jnp.float32),
          pltpu.SemaphoreType.DMA,
          pltpu.SemaphoreType.DMA,
      ],
      compiler_params=pltpu.CompilerParams(use_tc_tiling_on_sc=False),
  )(_body)
  return kern(x_flat, token_table, pos_table)


def kernel(x, token_table, pos_table):
  x_flat = x.reshape(-1).astype(jnp.int32)
  out = _tok_pos_embed(x_flat, token_table, pos_table)
  return out.reshape(BATCH, MAXLEN, EMBED)


# speedup vs baseline: 3.1825x; 1.0025x over previous
import functools

import jax
import jax.numpy as jnp
from jax import lax
from jax.experimental import pallas as pl
from jax.experimental.pallas import tpu as pltpu
from jax.experimental.pallas import tpu_sc as plsc

BATCH = 1024
MAXLEN = 200
EMBED = 64
LANES = 16

NUM_CORES = 2
NUM_SUBCORES = 16
NW = NUM_CORES * NUM_SUBCORES

ROWS_PER_W = BATCH // NW
CHUNK = MAXLEN
N_PER_W = ROWS_PER_W * MAXLEN


def _body(x_hbm, tok_hbm, pos_hbm, out_hbm,
          idx_v, pos_sh, rows0, rows1, sem0, sem1):
  sid = lax.axis_index("s")
  wid = sid * NUM_CORES + lax.axis_index("c")
  row0 = wid * ROWS_PER_W

  pltpu.sync_copy(x_hbm.at[pl.ds(row0, ROWS_PER_W), :], idx_v)

  @pl.when(sid == 0)
  def _():
    pltpu.sync_copy(pos_hbm, pos_sh)

  plsc.subcore_barrier()

  rows = (rows0, rows1)
  sems = (sem0, sem1)

  def start(c, b):
    pltpu.sync_copy(pos_sh, rows[b])
    pltpu.async_copy(
        tok_hbm.at[idx_v.at[c]], rows[b], sems[b], add=True)

  def wait(b):
    pltpu.make_async_copy(tok_hbm.at[pl.ds(0, CHUNK)], rows[b], sems[b]).wait()

  start(0, 0)
  start(1, 1)

  def outer(c0, _):
    for b in range(2):
      c = c0 + b
      wait(b)
      pltpu.sync_copy(rows[b], out_hbm.at[row0 + c])

      @pl.when(c + 2 < ROWS_PER_W)
      def _():
        start(c + 2, b)
    return 0

  lax.fori_loop(0, ROWS_PER_W // 2, lambda i, s: outer(i * 2, s), 0,
                unroll=False)


@jax.jit
def _tok_pos_embed(x, token_table, pos_table):
  mesh = plsc.VectorSubcoreMesh(core_axis_name="c", subcore_axis_name="s")
  kern = functools.partial(
      pl.kernel,
      out_type=jax.ShapeDtypeStruct((BATCH, MAXLEN, EMBED), jnp.float32),
      mesh=mesh,
      scratch_types=[
          pltpu.VMEM((ROWS_PER_W, MAXLEN), jnp.int32),
          pltpu.VMEM_SHARED((MAXLEN, EMBED), jnp.float32),
          pltpu.VMEM((CHUNK, EMBED), jnp.float32),
          pltpu.VMEM((CHUNK, EMBED), jnp.float32),
          pltpu.SemaphoreType.DMA,
          pltpu.SemaphoreType.DMA,
      ],
      compiler_params=pltpu.CompilerParams(use_tc_tiling_on_sc=False),
  )(_body)
  return kern(x, token_table, pos_table)


def kernel(x, token_table, pos_table):
  return _tok_pos_embed(x.astype(jnp.int32), token_table, pos_table)


# speedup vs baseline: 6.6566x; 2.0916x over previous
import functools

import jax
import jax.numpy as jnp
from jax import lax
from jax.experimental import pallas as pl
from jax.experimental.pallas import tpu as pltpu
from jax.experimental.pallas import tpu_sc as plsc

BATCH = 1024
MAXLEN = 200
EMBED = 64
LANES = 16

NUM_CORES = 2
NUM_SUBCORES = 16
NW = NUM_CORES * NUM_SUBCORES

NBG = BATCH // 128
NLQ = NW // NBG
LQ = MAXLEN // NLQ
TPAD = 129


def _body(xt_hbm, tok_hbm, pos_hbm, t5_hbm,
          idx_v, pos_v, grows0, grows1, tbuf0, tbuf1,
          g0, g1, s0, s1):
  wid = lax.axis_index("s") * NUM_CORES + lax.axis_index("c")
  bg = wid % NBG
  lq = wid // NBG
  l0 = lq * LQ

  pltpu.sync_copy(xt_hbm.at[pl.ds(l0, LQ), pl.ds(bg * 128, 128)], idx_v)
  pltpu.sync_copy(pos_hbm.at[pl.ds(l0, LQ)], pos_v)

  grows = (grows0, grows1)
  tbufs = (tbuf0, tbuf1)
  gsem = (g0, g1)
  ssem = (s0, s1)

  iota = lax.iota(jnp.int32, LANES)
  e_idx = [iota + LANES * j for j in range(EMBED // LANES)]

  def start_gather(i, p):
    pltpu.async_copy(tok_hbm.at[idx_v.at[i]], grows[p], gsem[p])

  def wait_gather(p):
    pltpu.make_async_copy(tok_hbm.at[pl.ds(0, 128)], grows[p], gsem[p]).wait()

  def drain_stores(p):
    pltpu.make_async_copy(tok_hbm.at[pl.ds(0, 128)], grows[p], ssem[p]).wait()

  start_gather(0, 0)
  start_gather(1, 1)

  def step(i, p):
    wait_gather(p)

    @pl.when(i + 2 < LQ)
    def _():
      start_gather(i + 2, p)

    @pl.when(i >= 2)
    def _():
      drain_stores(p)

    pvec = [pos_v[i, pl.ds(LANES * j, LANES)] for j in range(EMBED // LANES)]

    @plsc.parallel_loop(0, 128, 1, unroll=2)
    def _(b):
      bvec = jnp.full((LANES,), b, jnp.int32)
      for j in range(EMBED // LANES):
        val = grows[p][b, pl.ds(LANES * j, LANES)] + pvec[j]
        plsc.store_scatter(tbufs[p], [e_idx[j], bvec], val)

    for eh in range(EMBED // 8):
      pltpu.async_copy(
          tbufs[p].at[pl.ds(8 * eh, 8), pl.ds(0, 128)],
          t5_hbm.at[l0 + i, eh, bg], ssem[p])

  def pair(i0, _):
    step(i0, 0)
    step(i0 + 1, 1)
    return 0

  lax.fori_loop(0, LQ // 2, lambda k, s: pair(k * 2, s), 0, unroll=False)
  drain_stores(0)
  drain_stores(1)


@jax.jit
def _tok_pos_embed(xt, token_table, pos_table):
  mesh = plsc.VectorSubcoreMesh(core_axis_name="c", subcore_axis_name="s")
  kern = functools.partial(
      pl.kernel,
      out_type=jax.ShapeDtypeStruct((MAXLEN, 8, NBG, 8, 128), jnp.float32),
      mesh=mesh,
      scratch_types=[
          pltpu.VMEM((LQ, 128), jnp.int32),
          pltpu.VMEM((LQ, EMBED), jnp.float32),
          pltpu.VMEM((128, EMBED), jnp.float32),
          pltpu.VMEM((128, EMBED), jnp.float32),
          pltpu.VMEM((EMBED, TPAD), jnp.float32),
          pltpu.VMEM((EMBED, TPAD), jnp.float32),
          pltpu.SemaphoreType.DMA,
          pltpu.SemaphoreType.DMA,
          pltpu.SemaphoreType.DMA,
          pltpu.SemaphoreType.DMA,
      ],
      compiler_params=pltpu.CompilerParams(
          use_tc_tiling_on_sc=False, needs_layout_passes=False),
  )(_body)
  return kern(xt, token_table, pos_table)


def kernel(x, token_table, pos_table):
  t5 = _tok_pos_embed(x.T.astype(jnp.int32), token_table, pos_table)
  return t5.transpose(2, 4, 0, 1, 3).reshape(BATCH, MAXLEN, EMBED)
